# baseline (speedup 1.0000x reference)
import functools

import jax
import jax.numpy as jnp
from jax import lax
from jax.experimental import pallas as pl
from jax.experimental.pallas import tpu as pltpu
from jax.experimental.pallas import tpu_sc as plsc

_NC, _NS = 2, 16
_NW = _NC * _NS
_CH = 128
_BE = 1024
_BN = 2000


def _ln(x, g, b):
    m = jnp.mean(x, -1, keepdims=True)
    xc = x - m
    v = jnp.mean(xc * xc, -1, keepdims=True)
    return xc * lax.rsqrt(v + 1e-5) * g + b


def _make_gather(e_pad, d, n_acc, with_counts, dtype=jnp.bfloat16):
    per = e_pad // _NW
    steps = per // _CH
    pairs = steps // 2
    rows_t = n_acc // _NS
    n_z = rows_t // _CH + 1
    z_last = rows_t - (n_z - 1) * _CH
    mesh = plsc.VectorSubcoreMesh(core_axis_name="c", subcore_axis_name="s")

    out_type = [jax.ShapeDtypeStruct((e_pad, d), dtype),
                jax.ShapeDtypeStruct((e_pad, d), dtype)]
    scratch = [pltpu.VMEM((steps, _CH), jnp.int32),
               pltpu.VMEM((steps, _CH), jnp.int32),
               pltpu.VMEM((_CH, d), dtype),
               pltpu.VMEM((_CH, d), dtype),
               pltpu.VMEM((_CH, d), dtype),
               pltpu.VMEM((_CH, d), dtype)] + \
              [pltpu.SemaphoreType.DMA] * 5
    if with_counts:
        out_type += [jax.ShapeDtypeStruct((n_acc, 16), jnp.float32),
                     jax.ShapeDtypeStruct((n_acc, 16), jnp.float32)]
        scratch += [pltpu.VMEM((_CH, 16), jnp.float32),
                    pltpu.VMEM((_CH, 16), jnp.float32),
                    pltpu.VMEM_SHARED((n_acc, 16), jnp.float32),
                    pltpu.SemaphoreType.DMA]

    @functools.partial(
        pl.kernel,
        mesh=mesh,
        compiler_params=pltpu.CompilerParams(use_tc_tiling_on_sc=False),
        out_type=out_type,
        scratch_types=scratch,
    )
    def gather_k(x_hbm, row_hbm, col_hbm, grow_hbm, gcol_hbm, *rest):
        if with_counts:
            (c0_hbm, c1_hbm, idxr, idxc, bufr0, bufr1, bufc0, bufc1,
             semr0, semr1, semc0, semc1, semw,
             ones, z16, acc_c, semcnt) = rest
        else:
            (idxr, idxc, bufr0, bufr1, bufc0, bufc1,
             semr0, semr1, semc0, semc1, semw) = rest
        cid = lax.axis_index("c")
        sid = lax.axis_index("s")
        wid = sid * _NC + cid
        base = wid * per
        pltpu.sync_copy(row_hbm.at[wid], idxr)
        pltpu.sync_copy(col_hbm.at[wid], idxc)

        if with_counts:
            zero16 = jnp.zeros((16,), jnp.float32)
            one16 = jnp.ones((16,), jnp.float32)

            def fillrow(i, carry):
                z16[i] = zero16
                ones[i] = one16
                return carry

            lax.fori_loop(0, _CH, fillrow, 0)
            r0 = sid * rows_t
            for k in range(n_z):
                w = _CH if k < n_z - 1 else z_last
                if w > 0:
                    pltpu.sync_copy(z16.at[pl.ds(0, w)],
                                    acc_c.at[pl.ds(r0 + k * _CH, w)])
            plsc.subcore_barrier()

        def pair(p, carry):
            i0 = 2 * p
            i1 = i0 + 1
            if with_counts:
                k0 = pltpu.async_copy(ones, acc_c.at[idxc.at[i0]], semcnt,
                                      add=True)
                k1 = pltpu.async_copy(ones, acc_c.at[idxc.at[i1]], semcnt,
                                      add=True)
            g0r = pltpu.async_copy(x_hbm.at[idxr.at[i0]], bufr0, semr0)
            g0c = pltpu.async_copy(x_hbm.at[idxc.at[i0]], bufc0, semc0)
            g1r = pltpu.async_copy(x_hbm.at[idxr.at[i1]], bufr1, semr1)
            g1c = pltpu.async_copy(x_hbm.at[idxc.at[i1]], bufc1, semc1)
            g0r.wait()
            w0r = pltpu.async_copy(bufr0, grow_hbm.at[pl.ds(base + i0 * _CH,
                                                            _CH)], semw)
            g0c.wait()
            w0c = pltpu.async_copy(bufc0, gcol_hbm.at[pl.ds(base + i0 * _CH,
                                                            _CH)], semw)
            g1r.wait()
            w1r = pltpu.async_copy(bufr1, grow_hbm.at[pl.ds(base + i1 * _CH,
                                                            _CH)], semw)
            g1c.wait()
            w1c = pltpu.async_copy(bufc1, gcol_hbm.at[pl.ds(base + i1 * _CH,
                                                            _CH)], semw)
            w0r.wait()
            w0c.wait()
            w1r.wait()
            w1c.wait()
            if with_counts:
                k0.wait()
                k1.wait()
            return carry

        lax.fori_loop(0, pairs, pair, 0)

        if with_counts:
            plsc.subcore_barrier()
            r0 = sid * rows_t
            for k in range(n_z):
                w = _CH if k < n_z - 1 else z_last
                if w > 0:
                    rr = r0 + k * _CH
                    pltpu.sync_copy(acc_c.at[pl.ds(rr, w)],
                                    z16.at[pl.ds(0, w)])

                    @pl.when(cid == 0)
                    def _():
                        pltpu.sync_copy(z16.at[pl.ds(0, w)],
                                        c0_hbm.at[pl.ds(rr, w)])

                    @pl.when(cid == 1)
                    def _():
                        pltpu.sync_copy(z16.at[pl.ds(0, w)],
                                        c1_hbm.at[pl.ds(rr, w)])

    return gather_k


def _make_scatter(e_pad, d, n_acc):
    per = e_pad // _NW
    steps = per // _CH
    pairs = steps // 2
    rows_t = n_acc // _NS
    mesh = plsc.VectorSubcoreMesh(core_axis_name="c", subcore_axis_name="s")

    n_z = rows_t // _CH + 1
    z_last = rows_t - (n_z - 1) * _CH
    scratch = [pltpu.VMEM((steps, _CH), jnp.int32),
               pltpu.VMEM((_CH, d), jnp.float32),
               pltpu.VMEM((_CH, d), jnp.float32),
               pltpu.VMEM_SHARED((n_acc, d), jnp.float32)] + \
              [pltpu.SemaphoreType.DMA] * 4

    @functools.partial(
        pl.kernel,
        mesh=mesh,
        compiler_params=pltpu.CompilerParams(use_tc_tiling_on_sc=False),
        out_type=[jax.ShapeDtypeStruct((n_acc, d), jnp.float32),
                  jax.ShapeDtypeStruct((n_acc, d), jnp.float32)],
        scratch_types=scratch,
    )
    def scatter_k(m_hbm, col_hbm, s0_hbm, s1_hbm,
                  idx, vals0, vals1, acc_s, seml0, seml1, sems0, sems1):
        cid = lax.axis_index("c")
        sid = lax.axis_index("s")
        wid = sid * _NC + cid
        base = wid * per
        pltpu.sync_copy(col_hbm.at[wid], idx)

        zero16 = jnp.zeros((16,), jnp.float32)

        def zrow(i, carry):
            for k in range(d // 16):
                vals0[i, pl.ds(k * 16, 16)] = zero16
            return carry

        lax.fori_loop(0, _CH, zrow, 0)

        r0 = sid * rows_t
        for k in range(n_z):
            w = _CH if k < n_z - 1 else z_last
            if w > 0:
                pltpu.sync_copy(vals0.at[pl.ds(0, w)],
                                acc_s.at[pl.ds(r0 + k * _CH, w)])
        plsc.subcore_barrier()

        def pair(p, carry):
            i0 = 2 * p
            i1 = i0 + 1
            l0 = pltpu.async_copy(m_hbm.at[pl.ds(base + i0 * _CH, _CH)],
                                  vals0, seml0)
            l1 = pltpu.async_copy(m_hbm.at[pl.ds(base + i1 * _CH, _CH)],
                                  vals1, seml1)
            l0.wait()
            s0 = pltpu.async_copy(vals0, acc_s.at[idx.at[i0]], sems0,
                                  add=True)
            l1.wait()
            s1 = pltpu.async_copy(vals1, acc_s.at[idx.at[i1]], sems1,
                                  add=True)
            s0.wait()
            s1.wait()
            return carry

        lax.fori_loop(0, pairs, pair, 0)
        plsc.subcore_barrier()

        for k in range(n_z):
            w = _CH if k < n_z - 1 else z_last
            if w > 0:
                rr = r0 + k * _CH
                pltpu.sync_copy(acc_s.at[pl.ds(rr, w)], vals0.at[pl.ds(0, w)])

                @pl.when(cid == 0)
                def _():
                    pltpu.sync_copy(vals0.at[pl.ds(0, w)],
                                    s0_hbm.at[pl.ds(rr, w)])

                @pl.when(cid == 1)
                def _():
                    pltpu.sync_copy(vals0.at[pl.ds(0, w)],
                                    s1_hbm.at[pl.ds(rr, w)])

    return scatter_k


def _edge_body(grow_ref, gcol_ref, ea_ref, w_ref, c_ref, eout_ref, mout_ref):
    src = grow_ref[...].astype(jnp.float32)
    dst = gcol_ref[...].astype(jnp.float32)
    ea = ea_ref[...]
    c = c_ref[...]

    def dot(a, wi):
        return jnp.dot(a, w_ref[wi], preferred_element_type=jnp.float32)

    z = dot(src, 0) + dot(dst, 1) + dot(ea, 2) + c[0:1]
    h = _ln(jnp.maximum(z, 0.0), c[1:2], c[2:3])
    h = _ln(jnp.maximum(dot(h, 3) + c[3:4], 0.0), c[4:5], c[5:6])
    e_new = _ln(dot(h, 4) + c[6:7], c[7:8], c[8:9]) + ea
    eout_ref[...] = e_new
    zm = dot(src, 5) + dot(e_new, 6) + c[9:10]
    mout_ref[...] = _ln(jnp.maximum(zm, 0.0), c[10:11], c[11:12])


def _make_edge(e_pad, d):
    grid = (e_pad // _BE,)
    blk = lambda i: (i, 0)
    fixed3 = lambda i: (0, 0, 0)
    fixed2 = lambda i: (0, 0)
    return pl.pallas_call(
        _edge_body,
        grid=grid,
        in_specs=[pl.BlockSpec((_BE, d), blk),
                  pl.BlockSpec((_BE, d), blk),
                  pl.BlockSpec((_BE, d), blk),
                  pl.BlockSpec((7, d, d), fixed3),
                  pl.BlockSpec((12, d), fixed2)],
        out_specs=[pl.BlockSpec((_BE, d), blk),
                   pl.BlockSpec((_BE, d), blk)],
        out_shape=[jax.ShapeDtypeStruct((e_pad, d), jnp.float32),
                   jax.ShapeDtypeStruct((e_pad, d), jnp.float32)],
    )


def _node_body(x_ref, s0_ref, s1_ref, c0_ref, c1_ref, w_ref, c_ref, out_ref):
    x = x_ref[...]
    s = s0_ref[...] + s1_ref[...]
    cnt = c0_ref[...] + c1_ref[...]
    agg = s / jnp.maximum(cnt[:, 0:1], 1.0)
    c = c_ref[...]

    def dot(a, wi):
        return jnp.dot(a, w_ref[wi], preferred_element_type=jnp.float32)

    z = dot(x, 0) + dot(agg, 1) + c[0:1]
    h = _ln(jnp.maximum(z, 0.0), c[1:2], c[2:3])
    out_ref[...] = _ln(dot(h, 2) + c[3:4], c[4:5], c[5:6]) + x


def _make_node(n, d, n_acc):
    grid = (n // _BN,)
    blk = lambda i: (i, 0)
    fixed3 = lambda i: (0, 0, 0)
    fixed2 = lambda i: (0, 0)
    return pl.pallas_call(
        _node_body,
        grid=grid,
        in_specs=[pl.BlockSpec((_BN, d), blk),
                  pl.BlockSpec((_BN, d), blk),
                  pl.BlockSpec((_BN, d), blk),
                  pl.BlockSpec((_BN, 16), blk),
                  pl.BlockSpec((_BN, 16), blk),
                  pl.BlockSpec((3, d, d), fixed3),
                  pl.BlockSpec((6, d), fixed2)],
        out_specs=pl.BlockSpec((_BN, d), blk),
        out_shape=jax.ShapeDtypeStruct((n, d), jnp.float32),
    )


def kernel(X_h, edge_index, edge_attr_h, params):
    n, d = X_h.shape
    e = edge_index.shape[1]
    chunk = _NW * _CH * 2
    e_pad = ((e + chunk - 1) // chunk) * chunk
    n_acc = ((n + 1 + _NS * 8 - 1) // (_NS * 8)) * (_NS * 8)
    pad = e_pad - e
    steps = e_pad // _NW // _CH

    row3 = jnp.concatenate(
        [edge_index[0].astype(jnp.int32),
         jnp.full((pad,), n, jnp.int32)]).reshape(_NW, steps, _CH)
    col3 = jnp.concatenate(
        [edge_index[1].astype(jnp.int32),
         jnp.full((pad,), n, jnp.int32)]).reshape(_NW, steps, _CH)
    ea_pad = jnp.concatenate(
        [edge_attr_h, jnp.zeros((pad, d), jnp.float32)], axis=0)
    zrows = jnp.zeros((n_acc - n, d), jnp.bfloat16)

    gather1_f = _make_gather(e_pad, d, n_acc, True)
    gather2_f = _make_gather(e_pad, d, n_acc, False)
    scatter_f = _make_scatter(e_pad, d, n_acc)
    edge_f = _make_edge(e_pad, d)
    node_f = _make_node(n, d, n_acc)

    def fold_edge_params(pe, pn):
        w0 = pe[0]["W"]
        w = jnp.stack([w0[0:d], w0[d:2 * d], w0[2 * d:3 * d],
                       pe[1]["W"], pe[2]["W"],
                       pn[0]["W"][0:d], pn[0]["W"][d:2 * d]])
        c = jnp.stack([pe[0]["b"] + w0[3 * d], pe[0]["g"], pe[0]["beta"],
                       pe[1]["b"], pe[1]["g"], pe[1]["beta"],
                       pe[2]["b"], pe[2]["g"], pe[2]["beta"],
                       pn[0]["b"], pn[0]["g"], pn[0]["beta"]])
        return w, c

    def fold_node_params(pn):
        w1 = pn[1]["W"]
        w = jnp.stack([w1[0:d], w1[d:2 * d], pn[2]["W"]])
        c = jnp.stack([pn[1]["b"] + w1[2 * d], pn[1]["g"], pn[1]["beta"],
                       pn[2]["b"], pn[2]["g"], pn[2]["beta"]])
        return w, c

    def gn_layer(x, ea, pe, pn, counts):
        we, ce = fold_edge_params(pe, pn)
        wn, cn = fold_node_params(pn)
        x_pad = jnp.concatenate([x.astype(jnp.bfloat16), zrows], axis=0)
        if counts is None:
            grow, gcol, c0, c1 = gather1_f(x_pad, row3, col3)
        else:
            c0, c1 = counts
            grow, gcol = gather2_f(x_pad, row3, col3)
        e_new, m = edge_f(grow, gcol, ea, we, ce)
        s0, s1 = scatter_f(m, col3)
        x_new = node_f(x, s0, s1, c0, c1, wn, cn)
        return x_new, e_new, (c0, c1)

    x1, ea1, cnt = gn_layer(X_h, ea_pad,
                            params["gn1_edge"], params["gn1_node"], None)
    x2, ea2, _ = gn_layer(x1, ea1,
                          params["gn2_edge"], params["gn2_node"], cnt)
    return (x2, ea2[:e], jnp.ones((1, 1), jnp.float32))


# speedup vs baseline: 1.2982x; 1.2982x over previous
import functools

import jax
import jax.numpy as jnp
from jax import lax
from jax.experimental import pallas as pl
from jax.experimental.pallas import tpu as pltpu
from jax.experimental.pallas import tpu_sc as plsc

_NC, _NS = 2, 16
_NW = _NC * _NS
_CH = 128
_BE = 1024
_BN = 2000


def _ln(x, g, b):
    m = jnp.mean(x, -1, keepdims=True)
    xc = x - m
    v = jnp.mean(xc * xc, -1, keepdims=True)
    return xc * lax.rsqrt(v + 1e-5) * g + b


def _make_gather(e_pad, d, n_acc, with_counts, dtype=jnp.bfloat16):
    per = e_pad // _NW
    steps = per // _CH
    pairs = steps // 2
    rows_t = n_acc // _NS
    n_z = rows_t // _CH + 1
    z_last = rows_t - (n_z - 1) * _CH
    mesh = plsc.VectorSubcoreMesh(core_axis_name="c", subcore_axis_name="s")

    out_type = [jax.ShapeDtypeStruct((e_pad, d), dtype),
                jax.ShapeDtypeStruct((e_pad, d), dtype)]
    scratch = [pltpu.VMEM((steps, _CH), jnp.int32),
               pltpu.VMEM((steps, _CH), jnp.int32),
               pltpu.VMEM((_CH, d), dtype),
               pltpu.VMEM((_CH, d), dtype),
               pltpu.VMEM((_CH, d), dtype),
               pltpu.VMEM((_CH, d), dtype),
               pltpu.VMEM_SHARED((n_acc, d), dtype)] + \
              [pltpu.SemaphoreType.DMA] * 5
    if with_counts:
        out_type += [jax.ShapeDtypeStruct((n_acc, 16), jnp.float32),
                     jax.ShapeDtypeStruct((n_acc, 16), jnp.float32)]
        scratch += [pltpu.VMEM((_CH, 16), jnp.float32),
                    pltpu.VMEM((_CH, 16), jnp.float32),
                    pltpu.VMEM_SHARED((n_acc, 16), jnp.float32),
                    pltpu.SemaphoreType.DMA]

    @functools.partial(
        pl.kernel,
        mesh=mesh,
        compiler_params=pltpu.CompilerParams(use_tc_tiling_on_sc=False),
        out_type=out_type,
        scratch_types=scratch,
    )
    def gather_k(x_hbm, row_hbm, col_hbm, grow_hbm, gcol_hbm, *rest):
        if with_counts:
            (c0_hbm, c1_hbm, idxr, idxc, bufr0, bufr1, bufc0, bufc1, acc_x,
             semr0, semr1, semc0, semc1, semw,
             ones, z16, acc_c, semcnt) = rest
        else:
            (idxr, idxc, bufr0, bufr1, bufc0, bufc1, acc_x,
             semr0, semr1, semc0, semc1, semw) = rest
        cid = lax.axis_index("c")
        sid = lax.axis_index("s")
        wid = sid * _NC + cid
        base = wid * per
        pltpu.sync_copy(row_hbm.at[wid], idxr)
        pltpu.sync_copy(col_hbm.at[wid], idxc)

        rx = sid * rows_t
        for k in range(n_z):
            w = _CH if k < n_z - 1 else z_last
            if w > 0:
                pltpu.sync_copy(x_hbm.at[pl.ds(rx + k * _CH, w)],
                                bufr0.at[pl.ds(0, w)])
                pltpu.sync_copy(bufr0.at[pl.ds(0, w)],
                                acc_x.at[pl.ds(rx + k * _CH, w)])
        plsc.subcore_barrier()

        if with_counts:
            zero16 = jnp.zeros((16,), jnp.float32)
            one16 = jnp.ones((16,), jnp.float32)

            def fillrow(i, carry):
                z16[i] = zero16
                ones[i] = one16
                return carry

            lax.fori_loop(0, _CH, fillrow, 0)
            r0 = sid * rows_t
            for k in range(n_z):
                w = _CH if k < n_z - 1 else z_last
                if w > 0:
                    pltpu.sync_copy(z16.at[pl.ds(0, w)],
                                    acc_c.at[pl.ds(r0 + k * _CH, w)])
            plsc.subcore_barrier()

        def pair(p, carry):
            i0 = 2 * p
            i1 = i0 + 1
            if with_counts:
                k0 = pltpu.async_copy(ones, acc_c.at[idxc.at[i0]], semcnt,
                                      add=True)
                k1 = pltpu.async_copy(ones, acc_c.at[idxc.at[i1]], semcnt,
                                      add=True)
            g0r = pltpu.async_copy(acc_x.at[idxr.at[i0]], bufr0, semr0)
            g0c = pltpu.async_copy(acc_x.at[idxc.at[i0]], bufc0, semc0)
            g1r = pltpu.async_copy(acc_x.at[idxr.at[i1]], bufr1, semr1)
            g1c = pltpu.async_copy(acc_x.at[idxc.at[i1]], bufc1, semc1)
            g0r.wait()
            w0r = pltpu.async_copy(bufr0, grow_hbm.at[pl.ds(base + i0 * _CH,
                                                            _CH)], semw)
            g0c.wait()
            w0c = pltpu.async_copy(bufc0, gcol_hbm.at[pl.ds(base + i0 * _CH,
                                                            _CH)], semw)
            g1r.wait()
            w1r = pltpu.async_copy(bufr1, grow_hbm.at[pl.ds(base + i1 * _CH,
                                                            _CH)], semw)
            g1c.wait()
            w1c = pltpu.async_copy(bufc1, gcol_hbm.at[pl.ds(base + i1 * _CH,
                                                            _CH)], semw)
            w0r.wait()
            w0c.wait()
            w1r.wait()
            w1c.wait()
            if with_counts:
                k0.wait()
                k1.wait()
            return carry

        lax.fori_loop(0, pairs, pair, 0)

        if with_counts:
            plsc.subcore_barrier()
            r0 = sid * rows_t
            for k in range(n_z):
                w = _CH if k < n_z - 1 else z_last
                if w > 0:
                    rr = r0 + k * _CH
                    pltpu.sync_copy(acc_c.at[pl.ds(rr, w)],
                                    z16.at[pl.ds(0, w)])

                    @pl.when(cid == 0)
                    def _():
                        pltpu.sync_copy(z16.at[pl.ds(0, w)],
                                        c0_hbm.at[pl.ds(rr, w)])

                    @pl.when(cid == 1)
                    def _():
                        pltpu.sync_copy(z16.at[pl.ds(0, w)],
                                        c1_hbm.at[pl.ds(rr, w)])

    return gather_k


def _make_scatter(e_pad, d, n_acc):
    per = e_pad // _NW
    steps = per // _CH
    pairs = steps // 2
    rows_t = n_acc // _NS
    mesh = plsc.VectorSubcoreMesh(core_axis_name="c", subcore_axis_name="s")

    n_z = rows_t // _CH + 1
    z_last = rows_t - (n_z - 1) * _CH
    scratch = [pltpu.VMEM((steps, _CH), jnp.int32),
               pltpu.VMEM((_CH, d), jnp.float32),
               pltpu.VMEM((_CH, d), jnp.float32),
               pltpu.VMEM_SHARED((n_acc, d), jnp.float32)] + \
              [pltpu.SemaphoreType.DMA] * 4

    @functools.partial(
        pl.kernel,
        mesh=mesh,
        compiler_params=pltpu.CompilerParams(use_tc_tiling_on_sc=False),
        out_type=[jax.ShapeDtypeStruct((n_acc, d), jnp.float32),
                  jax.ShapeDtypeStruct((n_acc, d), jnp.float32)],
        scratch_types=scratch,
    )
    def scatter_k(m_hbm, col_hbm, s0_hbm, s1_hbm,
                  idx, vals0, vals1, acc_s, seml0, seml1, sems0, sems1):
        cid = lax.axis_index("c")
        sid = lax.axis_index("s")
        wid = sid * _NC + cid
        base = wid * per
        pltpu.sync_copy(col_hbm.at[wid], idx)

        zero16 = jnp.zeros((16,), jnp.float32)

        def zrow(i, carry):
            for k in range(d // 16):
                vals0[i, pl.ds(k * 16, 16)] = zero16
            return carry

        lax.fori_loop(0, _CH, zrow, 0)

        r0 = sid * rows_t
        for k in range(n_z):
            w = _CH if k < n_z - 1 else z_last
            if w > 0:
                pltpu.sync_copy(vals0.at[pl.ds(0, w)],
                                acc_s.at[pl.ds(r0 + k * _CH, w)])
        plsc.subcore_barrier()

        def pair(p, carry):
            i0 = 2 * p
            i1 = i0 + 1
            l0 = pltpu.async_copy(m_hbm.at[pl.ds(base + i0 * _CH, _CH)],
                                  vals0, seml0)
            l1 = pltpu.async_copy(m_hbm.at[pl.ds(base + i1 * _CH, _CH)],
                                  vals1, seml1)
            l0.wait()
            s0 = pltpu.async_copy(vals0, acc_s.at[idx.at[i0]], sems0,
                                  add=True)
            l1.wait()
            s1 = pltpu.async_copy(vals1, acc_s.at[idx.at[i1]], sems1,
                                  add=True)
            s0.wait()
            s1.wait()
            return carry

        lax.fori_loop(0, pairs, pair, 0)
        plsc.subcore_barrier()

        for k in range(n_z):
            w = _CH if k < n_z - 1 else z_last
            if w > 0:
                rr = r0 + k * _CH
                pltpu.sync_copy(acc_s.at[pl.ds(rr, w)], vals0.at[pl.ds(0, w)])

                @pl.when(cid == 0)
                def _():
                    pltpu.sync_copy(vals0.at[pl.ds(0, w)],
                                    s0_hbm.at[pl.ds(rr, w)])

                @pl.when(cid == 1)
                def _():
                    pltpu.sync_copy(vals0.at[pl.ds(0, w)],
                                    s1_hbm.at[pl.ds(rr, w)])

    return scatter_k


def _edge_body(grow_ref, gcol_ref, ea_ref, w_ref, c_ref, eout_ref, mout_ref):
    src = grow_ref[...].astype(jnp.float32)
    dst = gcol_ref[...].astype(jnp.float32)
    ea = ea_ref[...]
    c = c_ref[...]

    def dot(a, wi):
        return jnp.dot(a, w_ref[wi], preferred_element_type=jnp.float32)

    z = dot(src, 0) + dot(dst, 1) + dot(ea, 2) + c[0:1]
    h = _ln(jnp.maximum(z, 0.0), c[1:2], c[2:3])
    h = _ln(jnp.maximum(dot(h, 3) + c[3:4], 0.0), c[4:5], c[5:6])
    e_new = _ln(dot(h, 4) + c[6:7], c[7:8], c[8:9]) + ea
    eout_ref[...] = e_new
    zm = dot(src, 5) + dot(e_new, 6) + c[9:10]
    mout_ref[...] = _ln(jnp.maximum(zm, 0.0), c[10:11], c[11:12])


def _make_edge(e_pad, d):
    grid = (e_pad // _BE,)
    blk = lambda i: (i, 0)
    fixed3 = lambda i: (0, 0, 0)
    fixed2 = lambda i: (0, 0)
    return pl.pallas_call(
        _edge_body,
        grid=grid,
        in_specs=[pl.BlockSpec((_BE, d), blk),
                  pl.BlockSpec((_BE, d), blk),
                  pl.BlockSpec((_BE, d), blk),
                  pl.BlockSpec((7, d, d), fixed3),
                  pl.BlockSpec((12, d), fixed2)],
        out_specs=[pl.BlockSpec((_BE, d), blk),
                   pl.BlockSpec((_BE, d), blk)],
        out_shape=[jax.ShapeDtypeStruct((e_pad, d), jnp.float32),
                   jax.ShapeDtypeStruct((e_pad, d), jnp.float32)],
    )


def _node_body(x_ref, s0_ref, s1_ref, c0_ref, c1_ref, w_ref, c_ref, out_ref):
    x = x_ref[...]
    s = s0_ref[...] + s1_ref[...]
    cnt = c0_ref[...] + c1_ref[...]
    agg = s / jnp.maximum(cnt[:, 0:1], 1.0)
    c = c_ref[...]

    def dot(a, wi):
        return jnp.dot(a, w_ref[wi], preferred_element_type=jnp.float32)

    z = dot(x, 0) + dot(agg, 1) + c[0:1]
    h = _ln(jnp.maximum(z, 0.0), c[1:2], c[2:3])
    out_ref[...] = _ln(dot(h, 2) + c[3:4], c[4:5], c[5:6]) + x


def _make_node(n, d, n_acc):
    grid = (n // _BN,)
    blk = lambda i: (i, 0)
    fixed3 = lambda i: (0, 0, 0)
    fixed2 = lambda i: (0, 0)
    return pl.pallas_call(
        _node_body,
        grid=grid,
        in_specs=[pl.BlockSpec((_BN, d), blk),
                  pl.BlockSpec((_BN, d), blk),
                  pl.BlockSpec((_BN, d), blk),
                  pl.BlockSpec((_BN, 16), blk),
                  pl.BlockSpec((_BN, 16), blk),
                  pl.BlockSpec((3, d, d), fixed3),
                  pl.BlockSpec((6, d), fixed2)],
        out_specs=pl.BlockSpec((_BN, d), blk),
        out_shape=jax.ShapeDtypeStruct((n, d), jnp.float32),
    )


def kernel(X_h, edge_index, edge_attr_h, params):
    n, d = X_h.shape
    e = edge_index.shape[1]
    chunk = _NW * _CH * 2
    e_pad = ((e + chunk - 1) // chunk) * chunk
    n_acc = ((n + 1 + _NS * 8 - 1) // (_NS * 8)) * (_NS * 8)
    pad = e_pad - e
    steps = e_pad // _NW // _CH

    row3 = jnp.concatenate(
        [edge_index[0].astype(jnp.int32),
         jnp.full((pad,), n, jnp.int32)]).reshape(_NW, steps, _CH)
    col3 = jnp.concatenate(
        [edge_index[1].astype(jnp.int32),
         jnp.full((pad,), n, jnp.int32)]).reshape(_NW, steps, _CH)
    ea_pad = jnp.concatenate(
        [edge_attr_h, jnp.zeros((pad, d), jnp.float32)], axis=0)
    zrows = jnp.zeros((n_acc - n, d), jnp.bfloat16)

    gather1_f = _make_gather(e_pad, d, n_acc, True)
    gather2_f = _make_gather(e_pad, d, n_acc, False)
    scatter_f = _make_scatter(e_pad, d, n_acc)
    edge_f = _make_edge(e_pad, d)
    node_f = _make_node(n, d, n_acc)

    def fold_edge_params(pe, pn):
        w0 = pe[0]["W"]
        w = jnp.stack([w0[0:d], w0[d:2 * d], w0[2 * d:3 * d],
                       pe[1]["W"], pe[2]["W"],
                       pn[0]["W"][0:d], pn[0]["W"][d:2 * d]])
        c = jnp.stack([pe[0]["b"] + w0[3 * d], pe[0]["g"], pe[0]["beta"],
                       pe[1]["b"], pe[1]["g"], pe[1]["beta"],
                       pe[2]["b"], pe[2]["g"], pe[2]["beta"],
                       pn[0]["b"], pn[0]["g"], pn[0]["beta"]])
        return w, c

    def fold_node_params(pn):
        w1 = pn[1]["W"]
        w = jnp.stack([w1[0:d], w1[d:2 * d], pn[2]["W"]])
        c = jnp.stack([pn[1]["b"] + w1[2 * d], pn[1]["g"], pn[1]["beta"],
                       pn[2]["b"], pn[2]["g"], pn[2]["beta"]])
        return w, c

    def gn_layer(x, ea, pe, pn, counts):
        we, ce = fold_edge_params(pe, pn)
        wn, cn = fold_node_params(pn)
        x_pad = jnp.concatenate([x.astype(jnp.bfloat16), zrows], axis=0)
        if counts is None:
            grow, gcol, c0, c1 = gather1_f(x_pad, row3, col3)
        else:
            c0, c1 = counts
            grow, gcol = gather2_f(x_pad, row3, col3)
        e_new, m = edge_f(grow, gcol, ea, we, ce)
        s0, s1 = scatter_f(m, col3)
        x_new = node_f(x, s0, s1, c0, c1, wn, cn)
        return x_new, e_new, (c0, c1)

    x1, ea1, cnt = gn_layer(X_h, ea_pad,
                            params["gn1_edge"], params["gn1_node"], None)
    x2, ea2, _ = gn_layer(x1, ea1,
                          params["gn2_edge"], params["gn2_node"], cnt)
    return (x2, ea2[:e], jnp.ones((1, 1), jnp.float32))


# speedup vs baseline: 1.4687x; 1.1313x over previous
import functools

import jax
import jax.numpy as jnp
from jax import lax
from jax.experimental import pallas as pl
from jax.experimental.pallas import tpu as pltpu
from jax.experimental.pallas import tpu_sc as plsc

_NC, _NS = 2, 16
_NW = _NC * _NS
_CH = 128
_BE = 2560
_BN = 2000


def _ln(x, g, b):
    m = jnp.mean(x, -1, keepdims=True)
    xc = x - m
    v = jnp.mean(xc * xc, -1, keepdims=True)
    return xc * lax.rsqrt(v + 1e-5) * g + b


def _make_gather(e_pad, d, n_acc, with_counts, dtype=jnp.bfloat16):
    per = e_pad // _NW
    steps = per // _CH
    pairs = steps // 2
    rows_t = n_acc // _NS
    n_z = rows_t // _CH + 1
    z_last = rows_t - (n_z - 1) * _CH
    mesh = plsc.VectorSubcoreMesh(core_axis_name="c", subcore_axis_name="s")

    out_type = [jax.ShapeDtypeStruct((e_pad, d), dtype),
                jax.ShapeDtypeStruct((e_pad, d), dtype)]
    scratch = [pltpu.VMEM((steps, _CH), jnp.int32),
               pltpu.VMEM((steps, _CH), jnp.int32),
               pltpu.VMEM((_CH, d), dtype),
               pltpu.VMEM((_CH, d), dtype),
               pltpu.VMEM((_CH, d), dtype),
               pltpu.VMEM((_CH, d), dtype),
               pltpu.VMEM_SHARED((n_acc, d), dtype)] + \
              [pltpu.SemaphoreType.DMA] * 5
    if with_counts:
        out_type += [jax.ShapeDtypeStruct((n_acc, 16), jnp.float32),
                     jax.ShapeDtypeStruct((n_acc, 16), jnp.float32)]
        scratch += [pltpu.VMEM((_CH, 16), jnp.float32),
                    pltpu.VMEM((_CH, 16), jnp.float32),
                    pltpu.VMEM_SHARED((n_acc, 16), jnp.float32),
                    pltpu.SemaphoreType.DMA]

    @functools.partial(
        pl.kernel,
        mesh=mesh,
        compiler_params=pltpu.CompilerParams(use_tc_tiling_on_sc=False),
        out_type=out_type,
        scratch_types=scratch,
    )
    def gather_k(x_hbm, row_hbm, col_hbm, grow_hbm, gcol_hbm, *rest):
        if with_counts:
            (c0_hbm, c1_hbm, idxr, idxc, bufr0, bufr1, bufc0, bufc1, acc_x,
             semr0, semr1, semc0, semc1, semw,
             ones, z16, acc_c, semcnt) = rest
        else:
            (idxr, idxc, bufr0, bufr1, bufc0, bufc1, acc_x,
             semr0, semr1, semc0, semc1, semw) = rest
        cid = lax.axis_index("c")
        sid = lax.axis_index("s")
        wid = sid * _NC + cid
        base = wid * per
        pltpu.sync_copy(row_hbm.at[wid], idxr)
        pltpu.sync_copy(col_hbm.at[wid], idxc)

        rx = sid * rows_t
        for k in range(n_z):
            w = _CH if k < n_z - 1 else z_last
            if w > 0:
                pltpu.sync_copy(x_hbm.at[pl.ds(rx + k * _CH, w)],
                                bufr0.at[pl.ds(0, w)])
                pltpu.sync_copy(bufr0.at[pl.ds(0, w)],
                                acc_x.at[pl.ds(rx + k * _CH, w)])
        plsc.subcore_barrier()

        if with_counts:
            zero16 = jnp.zeros((16,), jnp.float32)
            one16 = jnp.ones((16,), jnp.float32)

            def fillrow(i, carry):
                z16[i] = zero16
                ones[i] = one16
                return carry

            lax.fori_loop(0, _CH, fillrow, 0)
            r0 = sid * rows_t
            for k in range(n_z):
                w = _CH if k < n_z - 1 else z_last
                if w > 0:
                    pltpu.sync_copy(z16.at[pl.ds(0, w)],
                                    acc_c.at[pl.ds(r0 + k * _CH, w)])
            plsc.subcore_barrier()

        def pair(p, carry):
            i0 = 2 * p
            i1 = i0 + 1
            if with_counts:
                k0 = pltpu.async_copy(ones, acc_c.at[idxc.at[i0]], semcnt,
                                      add=True)
                k1 = pltpu.async_copy(ones, acc_c.at[idxc.at[i1]], semcnt,
                                      add=True)
            g0r = pltpu.async_copy(acc_x.at[idxr.at[i0]], bufr0, semr0)
            g0c = pltpu.async_copy(acc_x.at[idxc.at[i0]], bufc0, semc0)
            g1r = pltpu.async_copy(acc_x.at[idxr.at[i1]], bufr1, semr1)
            g1c = pltpu.async_copy(acc_x.at[idxc.at[i1]], bufc1, semc1)
            g0r.wait()
            w0r = pltpu.async_copy(bufr0, grow_hbm.at[pl.ds(base + i0 * _CH,
                                                            _CH)], semw)
            g0c.wait()
            w0c = pltpu.async_copy(bufc0, gcol_hbm.at[pl.ds(base + i0 * _CH,
                                                            _CH)], semw)
            g1r.wait()
            w1r = pltpu.async_copy(bufr1, grow_hbm.at[pl.ds(base + i1 * _CH,
                                                            _CH)], semw)
            g1c.wait()
            w1c = pltpu.async_copy(bufc1, gcol_hbm.at[pl.ds(base + i1 * _CH,
                                                            _CH)], semw)
            w0r.wait()
            w0c.wait()
            w1r.wait()
            w1c.wait()
            if with_counts:
                k0.wait()
                k1.wait()
            return carry

        lax.fori_loop(0, pairs, pair, 0)

        if with_counts:
            plsc.subcore_barrier()
            r0 = sid * rows_t
            for k in range(n_z):
                w = _CH if k < n_z - 1 else z_last
                if w > 0:
                    rr = r0 + k * _CH
                    pltpu.sync_copy(acc_c.at[pl.ds(rr, w)],
                                    z16.at[pl.ds(0, w)])

                    @pl.when(cid == 0)
                    def _():
                        pltpu.sync_copy(z16.at[pl.ds(0, w)],
                                        c0_hbm.at[pl.ds(rr, w)])

                    @pl.when(cid == 1)
                    def _():
                        pltpu.sync_copy(z16.at[pl.ds(0, w)],
                                        c1_hbm.at[pl.ds(rr, w)])

    return gather_k


def _make_scatter(e_pad, d, n_acc):
    per = e_pad // _NW
    steps = per // _CH
    pairs = steps // 2
    rows_t = n_acc // _NS
    mesh = plsc.VectorSubcoreMesh(core_axis_name="c", subcore_axis_name="s")

    n_z = rows_t // _CH + 1
    z_last = rows_t - (n_z - 1) * _CH
    scratch = [pltpu.VMEM((steps, _CH), jnp.int32),
               pltpu.VMEM((_CH, d), jnp.float32),
               pltpu.VMEM((_CH, d), jnp.float32),
               pltpu.VMEM_SHARED((n_acc, d), jnp.float32)] + \
              [pltpu.SemaphoreType.DMA] * 4

    @functools.partial(
        pl.kernel,
        mesh=mesh,
        compiler_params=pltpu.CompilerParams(use_tc_tiling_on_sc=False),
        out_type=[jax.ShapeDtypeStruct((n_acc, d), jnp.float32),
                  jax.ShapeDtypeStruct((n_acc, d), jnp.float32)],
        scratch_types=scratch,
    )
    def scatter_k(m_hbm, col_hbm, s0_hbm, s1_hbm,
                  idx, vals0, vals1, acc_s, seml0, seml1, sems0, sems1):
        cid = lax.axis_index("c")
        sid = lax.axis_index("s")
        wid = sid * _NC + cid
        base = wid * per
        pltpu.sync_copy(col_hbm.at[wid], idx)

        zero16 = jnp.zeros((16,), jnp.float32)

        def zrow(i, carry):
            for k in range(d // 16):
                vals0[i, pl.ds(k * 16, 16)] = zero16
            return carry

        lax.fori_loop(0, _CH, zrow, 0)

        r0 = sid * rows_t
        for k in range(n_z):
            w = _CH if k < n_z - 1 else z_last
            if w > 0:
                pltpu.sync_copy(vals0.at[pl.ds(0, w)],
                                acc_s.at[pl.ds(r0 + k * _CH, w)])
        plsc.subcore_barrier()

        def pair(p, carry):
            i0 = 2 * p
            i1 = i0 + 1
            l0 = pltpu.async_copy(m_hbm.at[pl.ds(base + i0 * _CH, _CH)],
                                  vals0, seml0)
            l1 = pltpu.async_copy(m_hbm.at[pl.ds(base + i1 * _CH, _CH)],
                                  vals1, seml1)
            l0.wait()
            s0 = pltpu.async_copy(vals0, acc_s.at[idx.at[i0]], sems0,
                                  add=True)
            l1.wait()
            s1 = pltpu.async_copy(vals1, acc_s.at[idx.at[i1]], sems1,
                                  add=True)
            s0.wait()
            s1.wait()
            return carry

        lax.fori_loop(0, pairs, pair, 0)
        plsc.subcore_barrier()

        for k in range(n_z):
            w = _CH if k < n_z - 1 else z_last
            if w > 0:
                rr = r0 + k * _CH
                pltpu.sync_copy(acc_s.at[pl.ds(rr, w)], vals0.at[pl.ds(0, w)])

                @pl.when(cid == 0)
                def _():
                    pltpu.sync_copy(vals0.at[pl.ds(0, w)],
                                    s0_hbm.at[pl.ds(rr, w)])

                @pl.when(cid == 1)
                def _():
                    pltpu.sync_copy(vals0.at[pl.ds(0, w)],
                                    s1_hbm.at[pl.ds(rr, w)])

    return scatter_k


def _edge_body(grow_ref, gcol_ref, ea_ref, wsrc_ref, w_ref, c_ref,
               eout_ref, mout_ref):
    d = grow_ref.shape[1]
    ea = ea_ref[...]
    c = c_ref[...]

    def dot(a, wi):
        return jnp.dot(a, w_ref[wi], preferred_element_type=jnp.float32)

    y = jnp.dot(grow_ref[...], wsrc_ref[...],
                preferred_element_type=jnp.float32)
    z = y[:, :d] + dot(gcol_ref[...], 0) + dot(ea.astype(jnp.bfloat16), 1) \
        + c[0:1]
    h = _ln(jnp.maximum(z, 0.0), c[1:2], c[2:3])
    h = _ln(jnp.maximum(dot(h.astype(jnp.bfloat16), 2) + c[3:4], 0.0),
            c[4:5], c[5:6])
    e_new = _ln(dot(h.astype(jnp.bfloat16), 3) + c[6:7], c[7:8], c[8:9]) + ea
    eout_ref[...] = e_new
    zm = y[:, d:] + dot(e_new.astype(jnp.bfloat16), 4) + c[9:10]
    mout_ref[...] = _ln(jnp.maximum(zm, 0.0), c[10:11], c[11:12])


def _make_edge(e_pad, d):
    grid = (e_pad // _BE,)
    blk = lambda i: (i, 0)
    fixed3 = lambda i: (0, 0, 0)
    fixed2 = lambda i: (0, 0)
    return pl.pallas_call(
        _edge_body,
        grid=grid,
        in_specs=[pl.BlockSpec((_BE, d), blk),
                  pl.BlockSpec((_BE, d), blk),
                  pl.BlockSpec((_BE, d), blk),
                  pl.BlockSpec((d, 2 * d), fixed2),
                  pl.BlockSpec((5, d, d), fixed3),
                  pl.BlockSpec((12, d), fixed2)],
        out_specs=[pl.BlockSpec((_BE, d), blk),
                   pl.BlockSpec((_BE, d), blk)],
        out_shape=[jax.ShapeDtypeStruct((e_pad, d), jnp.float32),
                   jax.ShapeDtypeStruct((e_pad, d), jnp.float32)],
    )


def _node_body(x_ref, s0_ref, s1_ref, c0_ref, c1_ref, w_ref, c_ref, out_ref):
    x = x_ref[...]
    s = s0_ref[...] + s1_ref[...]
    cnt = c0_ref[...] + c1_ref[...]
    agg = s / jnp.maximum(cnt[:, 0:1], 1.0)
    c = c_ref[...]

    def dot(a, wi):
        return jnp.dot(a, w_ref[wi], preferred_element_type=jnp.float32)

    z = dot(x, 0) + dot(agg, 1) + c[0:1]
    h = _ln(jnp.maximum(z, 0.0), c[1:2], c[2:3])
    out_ref[...] = _ln(dot(h, 2) + c[3:4], c[4:5], c[5:6]) + x


def _make_node(n, d, n_acc):
    grid = (n // _BN,)
    blk = lambda i: (i, 0)
    fixed3 = lambda i: (0, 0, 0)
    fixed2 = lambda i: (0, 0)
    return pl.pallas_call(
        _node_body,
        grid=grid,
        in_specs=[pl.BlockSpec((_BN, d), blk),
                  pl.BlockSpec((_BN, d), blk),
                  pl.BlockSpec((_BN, d), blk),
                  pl.BlockSpec((_BN, 16), blk),
                  pl.BlockSpec((_BN, 16), blk),
                  pl.BlockSpec((3, d, d), fixed3),
                  pl.BlockSpec((6, d), fixed2)],
        out_specs=pl.BlockSpec((_BN, d), blk),
        out_shape=jax.ShapeDtypeStruct((n, d), jnp.float32),
    )


def kernel(X_h, edge_index, edge_attr_h, params):
    n, d = X_h.shape
    e = edge_index.shape[1]
    chunk = _NW * _CH * 2
    e_pad = ((e + chunk - 1) // chunk) * chunk
    n_acc = ((n + 1 + _NS * 8 - 1) // (_NS * 8)) * (_NS * 8)
    pad = e_pad - e
    steps = e_pad // _NW // _CH

    row3 = jnp.concatenate(
        [edge_index[0].astype(jnp.int32),
         jnp.full((pad,), n, jnp.int32)]).reshape(_NW, steps, _CH)
    col3 = jnp.concatenate(
        [edge_index[1].astype(jnp.int32),
         jnp.full((pad,), n, jnp.int32)]).reshape(_NW, steps, _CH)
    zrows = jnp.zeros((n_acc - n, d), jnp.bfloat16)

    gather1_f = _make_gather(e_pad, d, n_acc, True)
    gather2_f = _make_gather(e_pad, d, n_acc, False)
    scatter_f = _make_scatter(e_pad, d, n_acc)
    edge_f = _make_edge(e_pad, d)
    node_f = _make_node(n, d, n_acc)

    def fold_edge_params(pe, pn):
        w0 = pe[0]["W"]
        wsrc = jnp.concatenate([w0[0:d], pn[0]["W"][0:d]],
                               axis=1).astype(jnp.bfloat16)
        w = jnp.stack([w0[d:2 * d], w0[2 * d:3 * d],
                       pe[1]["W"], pe[2]["W"],
                       pn[0]["W"][d:2 * d]]).astype(jnp.bfloat16)
        c = jnp.stack([pe[0]["b"] + w0[3 * d], pe[0]["g"], pe[0]["beta"],
                       pe[1]["b"], pe[1]["g"], pe[1]["beta"],
                       pe[2]["b"], pe[2]["g"], pe[2]["beta"],
                       pn[0]["b"], pn[0]["g"], pn[0]["beta"]])
        return wsrc, w, c

    def fold_node_params(pn):
        w1 = pn[1]["W"]
        w = jnp.stack([w1[0:d], w1[d:2 * d], pn[2]["W"]])
        c = jnp.stack([pn[1]["b"] + w1[2 * d], pn[1]["g"], pn[1]["beta"],
                       pn[2]["b"], pn[2]["g"], pn[2]["beta"]])
        return w, c

    def gn_layer(x, ea, pe, pn, counts):
        wsrc, we, ce = fold_edge_params(pe, pn)
        wn, cn = fold_node_params(pn)
        x_pad = jnp.concatenate([x.astype(jnp.bfloat16), zrows], axis=0)
        if counts is None:
            grow, gcol, c0, c1 = gather1_f(x_pad, row3, col3)
        else:
            c0, c1 = counts
            grow, gcol = gather2_f(x_pad, row3, col3)
        e_new, m = edge_f(grow, gcol, ea, wsrc, we, ce)
        s0, s1 = scatter_f(m, col3)
        x_new = node_f(x, s0, s1, c0, c1, wn, cn)
        return x_new, e_new, (c0, c1)

    x1, ea1, cnt = gn_layer(X_h, edge_attr_h,
                            params["gn1_edge"], params["gn1_node"], None)
    x2, ea2, _ = gn_layer(x1, ea1,
                          params["gn2_edge"], params["gn2_node"], cnt)
    return (x2, ea2[:e], jnp.ones((1, 1), jnp.float32))


# speedup vs baseline: 1.4736x; 1.0034x over previous
import functools

import jax
import jax.numpy as jnp
from jax import lax
from jax.experimental import pallas as pl
from jax.experimental.pallas import tpu as pltpu
from jax.experimental.pallas import tpu_sc as plsc

_NC, _NS = 2, 16
_NW = _NC * _NS
_CH = 128
_BE = 2560
_BN = 2000


def _ln(x, g, b):
    m = jnp.mean(x, -1, keepdims=True)
    xc = x - m
    v = jnp.mean(xc * xc, -1, keepdims=True)
    return xc * lax.rsqrt(v + 1e-5) * g + b


def _make_gather(e_pad, d, n_acc, dtype=jnp.bfloat16):
    per = e_pad // _NW
    steps = per // _CH
    pairs = steps // 2
    rows_t = n_acc // _NS
    n_z = rows_t // _CH + 1
    z_last = rows_t - (n_z - 1) * _CH
    mesh = plsc.VectorSubcoreMesh(core_axis_name="c", subcore_axis_name="s")

    out_type = [jax.ShapeDtypeStruct((e_pad, d), dtype),
                jax.ShapeDtypeStruct((e_pad, d), dtype)]
    scratch = [pltpu.VMEM((steps, _CH), jnp.int32),
               pltpu.VMEM((steps, _CH), jnp.int32),
               pltpu.VMEM((_CH, d), dtype),
               pltpu.VMEM((_CH, d), dtype),
               pltpu.VMEM((_CH, d), dtype),
               pltpu.VMEM((_CH, d), dtype),
               pltpu.VMEM_SHARED((n_acc, d), dtype)] + \
              [pltpu.SemaphoreType.DMA] * 5

    @functools.partial(
        pl.kernel,
        mesh=mesh,
        compiler_params=pltpu.CompilerParams(use_tc_tiling_on_sc=False),
        out_type=out_type,
        scratch_types=scratch,
    )
    def gather_k(x_hbm, row_hbm, col_hbm, grow_hbm, gcol_hbm,
                 idxr, idxc, bufr0, bufr1, bufc0, bufc1, acc_x,
                 semr0, semr1, semc0, semc1, semw):
        cid = lax.axis_index("c")
        sid = lax.axis_index("s")
        wid = sid * _NC + cid
        base = wid * per
        pltpu.sync_copy(row_hbm.at[wid], idxr)
        pltpu.sync_copy(col_hbm.at[wid], idxc)

        rx = sid * rows_t
        for k in range(n_z):
            w = _CH if k < n_z - 1 else z_last
            if w > 0:
                pltpu.sync_copy(x_hbm.at[pl.ds(rx + k * _CH, w)],
                                bufr0.at[pl.ds(0, w)])
                pltpu.sync_copy(bufr0.at[pl.ds(0, w)],
                                acc_x.at[pl.ds(rx + k * _CH, w)])
        plsc.subcore_barrier()

        def pair(p, carry):
            i0 = 2 * p
            i1 = i0 + 1
            g0r = pltpu.async_copy(acc_x.at[idxr.at[i0]], bufr0, semr0)
            g0c = pltpu.async_copy(acc_x.at[idxc.at[i0]], bufc0, semc0)
            g1r = pltpu.async_copy(acc_x.at[idxr.at[i1]], bufr1, semr1)
            g1c = pltpu.async_copy(acc_x.at[idxc.at[i1]], bufc1, semc1)
            g0r.wait()
            w0r = pltpu.async_copy(bufr0, grow_hbm.at[pl.ds(base + i0 * _CH,
                                                            _CH)], semw)
            g0c.wait()
            w0c = pltpu.async_copy(bufc0, gcol_hbm.at[pl.ds(base + i0 * _CH,
                                                            _CH)], semw)
            g1r.wait()
            w1r = pltpu.async_copy(bufr1, grow_hbm.at[pl.ds(base + i1 * _CH,
                                                            _CH)], semw)
            g1c.wait()
            w1c = pltpu.async_copy(bufc1, gcol_hbm.at[pl.ds(base + i1 * _CH,
                                                            _CH)], semw)
            w0r.wait()
            w0c.wait()
            w1r.wait()
            w1c.wait()
            return carry

        lax.fori_loop(0, pairs, pair, 0)

    return gather_k


def _make_counts(e_pad, n_acc):
    per = e_pad // _NW
    steps = per // _CH
    rows_t = n_acc // _NS
    n_z = rows_t // _CH + 1
    z_last = rows_t - (n_z - 1) * _CH
    mesh = plsc.VectorSubcoreMesh(core_axis_name="c", subcore_axis_name="s")

    @functools.partial(
        pl.kernel,
        mesh=mesh,
        compiler_params=pltpu.CompilerParams(use_tc_tiling_on_sc=False),
        out_type=[jax.ShapeDtypeStruct((n_acc, 16), jnp.float32),
                  jax.ShapeDtypeStruct((n_acc, 16), jnp.float32)],
        scratch_types=[pltpu.VMEM((steps, _CH), jnp.int32),
                       pltpu.VMEM((_CH, 16), jnp.float32),
                       pltpu.VMEM((_CH, 16), jnp.float32),
                       pltpu.VMEM_SHARED((n_acc, 16), jnp.float32),
                       pltpu.SemaphoreType.DMA],
    )
    def counts_k(col_hbm, c0_hbm, c1_hbm, idxc, ones, z16, acc_c, semcnt):
        cid = lax.axis_index("c")
        sid = lax.axis_index("s")
        wid = sid * _NC + cid
        pltpu.sync_copy(col_hbm.at[wid], idxc)
        zero16 = jnp.zeros((16,), jnp.float32)
        one16 = jnp.ones((16,), jnp.float32)

        def fillrow(i, carry):
            z16[i] = zero16
            ones[i] = one16
            return carry

        lax.fori_loop(0, _CH, fillrow, 0)
        r0 = sid * rows_t
        for k in range(n_z):
            w = _CH if k < n_z - 1 else z_last
            if w > 0:
                pltpu.sync_copy(z16.at[pl.ds(0, w)],
                                acc_c.at[pl.ds(r0 + k * _CH, w)])
        plsc.subcore_barrier()

        def pair(p, carry):
            k0 = pltpu.async_copy(ones, acc_c.at[idxc.at[2 * p]], semcnt,
                                  add=True)
            k1 = pltpu.async_copy(ones, acc_c.at[idxc.at[2 * p + 1]], semcnt,
                                  add=True)
            k0.wait()
            k1.wait()
            return carry

        lax.fori_loop(0, steps // 2, pair, 0)
        plsc.subcore_barrier()

        for k in range(n_z):
            w = _CH if k < n_z - 1 else z_last
            if w > 0:
                rr = r0 + k * _CH
                pltpu.sync_copy(acc_c.at[pl.ds(rr, w)], z16.at[pl.ds(0, w)])

                @pl.when(cid == 0)
                def _():
                    pltpu.sync_copy(z16.at[pl.ds(0, w)],
                                    c0_hbm.at[pl.ds(rr, w)])

                @pl.when(cid == 1)
                def _():
                    pltpu.sync_copy(z16.at[pl.ds(0, w)],
                                    c1_hbm.at[pl.ds(rr, w)])

    return counts_k


def _make_scatter(e_pad, d, n_acc):
    per = e_pad // _NW
    steps = per // _CH
    pairs = steps // 2
    rows_t = n_acc // _NS
    mesh = plsc.VectorSubcoreMesh(core_axis_name="c", subcore_axis_name="s")

    n_z = rows_t // _CH + 1
    z_last = rows_t - (n_z - 1) * _CH
    scratch = [pltpu.VMEM((steps, _CH), jnp.int32),
               pltpu.VMEM((_CH, d), jnp.float32),
               pltpu.VMEM((_CH, d), jnp.float32),
               pltpu.VMEM_SHARED((n_acc, d), jnp.float32)] + \
              [pltpu.SemaphoreType.DMA] * 4

    @functools.partial(
        pl.kernel,
        mesh=mesh,
        compiler_params=pltpu.CompilerParams(use_tc_tiling_on_sc=False),
        out_type=[jax.ShapeDtypeStruct((n_acc, d), jnp.float32),
                  jax.ShapeDtypeStruct((n_acc, d), jnp.float32)],
        scratch_types=scratch,
    )
    def scatter_k(m_hbm, col_hbm, s0_hbm, s1_hbm,
                  idx, vals0, vals1, acc_s, seml0, seml1, sems0, sems1):
        cid = lax.axis_index("c")
        sid = lax.axis_index("s")
        wid = sid * _NC + cid
        base = wid * per
        pltpu.sync_copy(col_hbm.at[wid], idx)

        zero16 = jnp.zeros((16,), jnp.float32)

        def zrow(i, carry):
            for k in range(d // 16):
                vals0[i, pl.ds(k * 16, 16)] = zero16
            return carry

        lax.fori_loop(0, _CH, zrow, 0)

        r0 = sid * rows_t
        for k in range(n_z):
            w = _CH if k < n_z - 1 else z_last
            if w > 0:
                pltpu.sync_copy(vals0.at[pl.ds(0, w)],
                                acc_s.at[pl.ds(r0 + k * _CH, w)])
        plsc.subcore_barrier()

        def pair(p, carry):
            i0 = 2 * p
            i1 = i0 + 1
            l0 = pltpu.async_copy(m_hbm.at[pl.ds(base + i0 * _CH, _CH)],
                                  vals0, seml0)
            l1 = pltpu.async_copy(m_hbm.at[pl.ds(base + i1 * _CH, _CH)],
                                  vals1, seml1)
            l0.wait()
            s0 = pltpu.async_copy(vals0, acc_s.at[idx.at[i0]], sems0,
                                  add=True)
            l1.wait()
            s1 = pltpu.async_copy(vals1, acc_s.at[idx.at[i1]], sems1,
                                  add=True)
            s0.wait()
            s1.wait()
            return carry

        lax.fori_loop(0, pairs, pair, 0)
        plsc.subcore_barrier()

        for k in range(n_z):
            w = _CH if k < n_z - 1 else z_last
            if w > 0:
                rr = r0 + k * _CH
                pltpu.sync_copy(acc_s.at[pl.ds(rr, w)], vals0.at[pl.ds(0, w)])

                @pl.when(cid == 0)
                def _():
                    pltpu.sync_copy(vals0.at[pl.ds(0, w)],
                                    s0_hbm.at[pl.ds(rr, w)])

                @pl.when(cid == 1)
                def _():
                    pltpu.sync_copy(vals0.at[pl.ds(0, w)],
                                    s1_hbm.at[pl.ds(rr, w)])

    return scatter_k


def _edge_body(grow_ref, gcol_ref, ea_ref, wsrc_ref, w_ref, c_ref,
               eout_ref, mout_ref):
    d = grow_ref.shape[1]
    ea = ea_ref[...]
    c = c_ref[...]

    def dot(a, wi):
        return jnp.dot(a, w_ref[wi], preferred_element_type=jnp.float32)

    y = jnp.dot(grow_ref[...], wsrc_ref[...],
                preferred_element_type=jnp.float32)
    z = y[:, :d] + dot(gcol_ref[...], 0) + dot(ea.astype(jnp.bfloat16), 1) \
        + c[0:1]
    h = _ln(jnp.maximum(z, 0.0), c[1:2], c[2:3])
    h = _ln(jnp.maximum(dot(h.astype(jnp.bfloat16), 2) + c[3:4], 0.0),
            c[4:5], c[5:6])
    e_new = _ln(dot(h.astype(jnp.bfloat16), 3) + c[6:7], c[7:8], c[8:9]) + ea
    eout_ref[...] = e_new
    zm = y[:, d:] + dot(e_new.astype(jnp.bfloat16), 4) + c[9:10]
    mout_ref[...] = _ln(jnp.maximum(zm, 0.0), c[10:11], c[11:12])


def _make_edge(e_pad, d):
    grid = (e_pad // _BE,)
    blk = lambda i: (i, 0)
    fixed3 = lambda i: (0, 0, 0)
    fixed2 = lambda i: (0, 0)
    return pl.pallas_call(
        _edge_body,
        grid=grid,
        in_specs=[pl.BlockSpec((_BE, d), blk),
                  pl.BlockSpec((_BE, d), blk),
                  pl.BlockSpec((_BE, d), blk),
                  pl.BlockSpec((d, 2 * d), fixed2),
                  pl.BlockSpec((5, d, d), fixed3),
                  pl.BlockSpec((12, d), fixed2)],
        out_specs=[pl.BlockSpec((_BE, d), blk),
                   pl.BlockSpec((_BE, d), blk)],
        out_shape=[jax.ShapeDtypeStruct((e_pad, d), jnp.float32),
                   jax.ShapeDtypeStruct((e_pad, d), jnp.float32)],
    )


def _node_body(x_ref, s0_ref, s1_ref, c0_ref, c1_ref, w_ref, c_ref, out_ref):
    x = x_ref[...]
    s = s0_ref[...] + s1_ref[...]
    cnt = c0_ref[...] + c1_ref[...]
    agg = s / jnp.maximum(cnt[:, 0:1], 1.0)
    c = c_ref[...]

    def dot(a, wi):
        return jnp.dot(a, w_ref[wi], preferred_element_type=jnp.float32)

    z = dot(x, 0) + dot(agg, 1) + c[0:1]
    h = _ln(jnp.maximum(z, 0.0), c[1:2], c[2:3])
    out_ref[...] = _ln(dot(h, 2) + c[3:4], c[4:5], c[5:6]) + x


def _make_node(n, d, n_acc):
    grid = (n // _BN,)
    blk = lambda i: (i, 0)
    fixed3 = lambda i: (0, 0, 0)
    fixed2 = lambda i: (0, 0)
    return pl.pallas_call(
        _node_body,
        grid=grid,
        in_specs=[pl.BlockSpec((_BN, d), blk),
                  pl.BlockSpec((_BN, d), blk),
                  pl.BlockSpec((_BN, d), blk),
                  pl.BlockSpec((_BN, 16), blk),
                  pl.BlockSpec((_BN, 16), blk),
                  pl.BlockSpec((3, d, d), fixed3),
                  pl.BlockSpec((6, d), fixed2)],
        out_specs=pl.BlockSpec((_BN, d), blk),
        out_shape=jax.ShapeDtypeStruct((n, d), jnp.float32),
    )


def kernel(X_h, edge_index, edge_attr_h, params):
    n, d = X_h.shape
    e = edge_index.shape[1]
    chunk = _NW * _CH * 2
    e_pad = ((e + chunk - 1) // chunk) * chunk
    n_acc = ((n + 1 + _NS * 16 - 1) // (_NS * 16)) * (_NS * 16)
    pad = e_pad - e
    steps = e_pad // _NW // _CH

    row3 = jnp.concatenate(
        [edge_index[0].astype(jnp.int32),
         jnp.full((pad,), n, jnp.int32)]).reshape(_NW, steps, _CH)
    col3 = jnp.concatenate(
        [edge_index[1].astype(jnp.int32),
         jnp.full((pad,), n, jnp.int32)]).reshape(_NW, steps, _CH)
    zrows = jnp.zeros((n_acc - n, d), jnp.bfloat16)

    gather_f = _make_gather(e_pad, d, n_acc)
    counts_f = _make_counts(e_pad, n_acc)
    scatter_f = _make_scatter(e_pad, d, n_acc)
    edge_f = _make_edge(e_pad, d)
    node_f = _make_node(n, d, n_acc)

    def fold_edge_params(pe, pn):
        w0 = pe[0]["W"]
        wsrc = jnp.concatenate([w0[0:d], pn[0]["W"][0:d]],
                               axis=1).astype(jnp.bfloat16)
        w = jnp.stack([w0[d:2 * d], w0[2 * d:3 * d],
                       pe[1]["W"], pe[2]["W"],
                       pn[0]["W"][d:2 * d]]).astype(jnp.bfloat16)
        c = jnp.stack([pe[0]["b"] + w0[3 * d], pe[0]["g"], pe[0]["beta"],
                       pe[1]["b"], pe[1]["g"], pe[1]["beta"],
                       pe[2]["b"], pe[2]["g"], pe[2]["beta"],
                       pn[0]["b"], pn[0]["g"], pn[0]["beta"]])
        return wsrc, w, c

    def fold_node_params(pn):
        w1 = pn[1]["W"]
        w = jnp.stack([w1[0:d], w1[d:2 * d], pn[2]["W"]])
        c = jnp.stack([pn[1]["b"] + w1[2 * d], pn[1]["g"], pn[1]["beta"],
                       pn[2]["b"], pn[2]["g"], pn[2]["beta"]])
        return w, c

    def gn_layer(x, ea, pe, pn, counts):
        wsrc, we, ce = fold_edge_params(pe, pn)
        wn, cn = fold_node_params(pn)
        x_pad = jnp.concatenate([x.astype(jnp.bfloat16), zrows], axis=0)
        if counts is None:
            c0, c1 = counts_f(col3)
        else:
            c0, c1 = counts
        grow, gcol = gather_f(x_pad, row3, col3)
        e_new, m = edge_f(grow, gcol, ea, wsrc, we, ce)
        s0, s1 = scatter_f(m, col3)
        x_new = node_f(x, s0, s1, c0, c1, wn, cn)
        return x_new, e_new, (c0, c1)

    x1, ea1, cnt = gn_layer(X_h, edge_attr_h,
                            params["gn1_edge"], params["gn1_node"], None)
    x2, ea2, _ = gn_layer(x1, ea1,
                          params["gn2_edge"], params["gn2_node"], cnt)
    return (x2, ea2[:e], jnp.ones((1, 1), jnp.float32))
